# baseline (speedup 1.0000x reference)
import functools

import jax
import jax.numpy as jnp
from jax import lax
from jax.experimental import pallas as pl
from jax.experimental.pallas import tpu as pltpu
from jax.experimental.pallas import tpu_sc as plsc

N_NODES = 10000
NPAD = 10240
D = 128
E_EDGES = 320000
NC = 2
NS = 16
NW = NC * NS
EPW = E_EDGES // NW
SB = 2000
NBLK = EPW // SB
BCHUNKS = SB // 16
NBUF = 5
BOUT = BCHUNKS // NBUF
EPS = E_EDGES // NS
DEG_K = 25
DEG_OUT = BCHUNKS // DEG_K
ROWS_PT = NPAD // NS
ROW_BLK = 1000


def _matmul_body(x_ref, w_ref, o_ref):
    o_ref[...] = jnp.dot(x_ref[...], w_ref[...],
                         preferred_element_type=jnp.float32)


def _xw(x, W):
    return pl.pallas_call(
        _matmul_body,
        grid=(N_NODES // ROW_BLK,),
        in_specs=[
            pl.BlockSpec((ROW_BLK, D), lambda i: (i, 0)),
            pl.BlockSpec((D, D), lambda i: (0, 0)),
        ],
        out_specs=pl.BlockSpec((ROW_BLK, D), lambda i: (i, 0)),
        out_shape=jax.ShapeDtypeStruct((N_NODES, D), jnp.float32),
    )(x, W)


def _epilogue_body(a0_ref, a1_ref, dinv_ref, xw_ref, b_ref, o_ref):
    dinv = dinv_ref[...]
    acc = a0_ref[...] + a1_ref[...]
    out = dinv * acc + (dinv * dinv) * xw_ref[...] + b_ref[...]
    o_ref[...] = jnp.maximum(out, 0.0)


def _epilogue(a0, a1, dinv, xw, b):
    blk = lambda i: (i, 0)
    return pl.pallas_call(
        _epilogue_body,
        grid=(N_NODES // ROW_BLK,),
        in_specs=[
            pl.BlockSpec((ROW_BLK, D), blk),
            pl.BlockSpec((ROW_BLK, D), blk),
            pl.BlockSpec((ROW_BLK, 1), blk),
            pl.BlockSpec((ROW_BLK, D), blk),
            pl.BlockSpec((1, D), lambda i: (0, 0)),
        ],
        out_specs=pl.BlockSpec((ROW_BLK, D), blk),
        out_shape=jax.ShapeDtypeStruct((N_NODES, D), jnp.float32),
    )(a0, a1, dinv.reshape(N_NODES, 1), xw, b.reshape(1, D))


def _splat_lane(a, e):
    return lax.gather(
        a, jnp.full((16, 1), e, jnp.int32),
        lax.GatherDimensionNumbers(
            offset_dims=(), collapsed_slice_dims=(0,),
            start_index_map=(0,)),
        (1,), mode=lax.GatherScatterMode.PROMISE_IN_BOUNDS)


def _newton_rsqrt(d):
    i = lax.bitcast_convert_type(d, jnp.int32)
    i = jnp.int32(0x5F3759DF) - lax.shift_right_logical(i, 1)
    y = lax.bitcast_convert_type(i, jnp.float32)
    half_d = 0.5 * d
    for _ in range(3):
        y = y * (1.5 - half_d * y * y)
    return y


def _main_body(xw_hbm, src_hbm, dst_hbm, w_hbm, z1_hbm, z2_hbm,
               acc_out, dinv_out,
               srcb_v, dstb_v, wb_v, dinv_v, degl_v,
               gbuf, sbuf,
               acc_sp, dinv_sp,
               gsem, ssem, degsem):
    c = lax.axis_index("c")
    s = lax.axis_index("s")
    wid = s * NC + c
    base = s * ROWS_PT

    pltpu.sync_copy(z2_hbm, acc_sp.at[pl.ds(base, ROWS_PT)])
    pltpu.sync_copy(z1_hbm, dinv_sp.at[pl.ds(base, ROWS_PT)])
    plsc.subcore_barrier()

    @pl.loop(0, EPS // SB)
    def _(blk):
        off = s * EPS + blk * SB
        pltpu.sync_copy(dst_hbm.at[pl.ds(off, SB)], dstb_v)
        pltpu.sync_copy(w_hbm.at[pl.ds(off, SB)], wb_v)

        @pl.loop(0, DEG_OUT)
        def _(j0):
            for jj in range(DEG_K):
                j = j0 * DEG_K + jj
                dv = dstb_v[pl.ds(j * 16, 16)]
                pltpu.async_copy(wb_v.at[pl.ds(j * 16, 16)],
                                 dinv_sp.at[dv], degsem, add=True)
            for jj in range(DEG_K):
                j = j0 * DEG_K + jj
                dv = dstb_v[pl.ds(j * 16, 16)]
                pltpu.make_async_copy(wb_v.at[pl.ds(j * 16, 16)],
                                      dinv_sp.at[dv], degsem).wait()
    plsc.subcore_barrier()

    pltpu.sync_copy(dinv_sp.at[pl.ds(base, ROWS_PT)], degl_v)

    @pl.loop(0, ROWS_PT // 16)
    def _(i):
        sl = pl.ds(i * 16, 16)
        d = degl_v[sl] + 1.0
        degl_v[sl] = _newton_rsqrt(d)

    pltpu.sync_copy(degl_v, dinv_sp.at[pl.ds(base, ROWS_PT)])

    @pl.when(c == 0)
    def _():
        pltpu.sync_copy(degl_v, dinv_out.at[pl.ds(base, ROWS_PT)])
    plsc.subcore_barrier()
    pltpu.sync_copy(dinv_sp, dinv_v)

    def fire_gather(g, b):
        sv = srcb_v[pl.ds(g * 16, 16)]
        pltpu.async_copy(xw_hbm.at[sv], gbuf.at[b], gsem.at[b])

    def wait_gather(g, b):
        sv = srcb_v[pl.ds(g * 16, 16)]
        pltpu.make_async_copy(xw_hbm.at[sv], gbuf.at[b],
                              gsem.at[b]).wait()

    def fire_scatter(g, b):
        dv = dstb_v[pl.ds(g * 16, 16)]
        pltpu.async_copy(sbuf.at[b], acc_sp.at[dv], ssem.at[b], add=True)

    def wait_scatter(g, b):
        dv = dstb_v[pl.ds(g * 16, 16)]
        pltpu.make_async_copy(sbuf.at[b], acc_sp.at[dv],
                              ssem.at[b]).wait()

    @pl.loop(0, NBLK)
    def _(blk):
        off = wid * EPW + blk * SB
        pltpu.sync_copy(src_hbm.at[pl.ds(off, SB)], srcb_v)
        pltpu.sync_copy(dst_hbm.at[pl.ds(off, SB)], dstb_v)
        pltpu.sync_copy(w_hbm.at[pl.ds(off, SB)], wb_v)

        for b in range(NBUF):
            fire_gather(b, b)

        @pl.loop(0, BOUT)
        def _(i):
            for b in range(NBUF):
                g = i * NBUF + b
                wait_gather(g, b)
                sv = srcb_v[pl.ds(g * 16, 16)]
                a = wb_v[pl.ds(g * 16, 16)] * plsc.load_gather(
                    dinv_v, [sv])

                @pl.when(i > 0)
                def _():
                    wait_scatter(g - NBUF, b)

                for e in range(16):
                    ab = _splat_lane(a, e)
                    for cc in range(8):
                        sbuf[b, e, pl.ds(cc * 16, 16)] = (
                            gbuf[b, e, pl.ds(cc * 16, 16)] * ab)
                fire_scatter(g, b)

                @pl.when(i < BOUT - 1)
                def _():
                    fire_gather(g + NBUF, b)

        for b in range(NBUF):
            wait_scatter(BCHUNKS - NBUF + b, b)

    plsc.subcore_barrier()
    pltpu.sync_copy(acc_sp.at[pl.ds(base, ROWS_PT)],
                    acc_out.at[c, pl.ds(base, ROWS_PT)])


_main_kernel = functools.partial(
    pl.kernel,
    out_type=(
        jax.ShapeDtypeStruct((NC, NPAD, D), jnp.float32),
        jax.ShapeDtypeStruct((NPAD,), jnp.float32),
    ),
    mesh=plsc.VectorSubcoreMesh(core_axis_name="c", subcore_axis_name="s"),
    compiler_params=pltpu.CompilerParams(needs_layout_passes=False),
    scratch_types=(
        pltpu.VMEM((SB,), jnp.int32),
        pltpu.VMEM((SB,), jnp.int32),
        pltpu.VMEM((SB,), jnp.float32),
        pltpu.VMEM((NPAD,), jnp.float32),
        pltpu.VMEM((ROWS_PT,), jnp.float32),
        pltpu.VMEM((NBUF, 16, D), jnp.float32),
        pltpu.VMEM((NBUF, 16, D), jnp.float32),
        pltpu.VMEM_SHARED((NPAD, D), jnp.float32),
        pltpu.VMEM_SHARED((NPAD,), jnp.float32),
        pltpu.SemaphoreType.DMA((NBUF,)),
        pltpu.SemaphoreType.DMA((NBUF,)),
        pltpu.SemaphoreType.DMA,
    ),
)(_main_body)


def kernel(x, edge_index, edge_attr, W, b):
    src = edge_index[0]
    dst = edge_index[1]
    xw = _xw(x, W)
    z1 = jnp.zeros((ROWS_PT,), jnp.float32)
    z2 = jnp.zeros((ROWS_PT, D), jnp.float32)
    acc, dinv = _main_kernel(xw, src, dst, edge_attr, z1, z2)
    return _epilogue(acc[0, :N_NODES], acc[1, :N_NODES],
                     dinv[:N_NODES], xw, b)


# speedup vs baseline: 1.0666x; 1.0666x over previous
import functools

import jax
import jax.numpy as jnp
from jax import lax
from jax.experimental import pallas as pl
from jax.experimental.pallas import tpu as pltpu
from jax.experimental.pallas import tpu_sc as plsc

N_NODES = 10000
NPAD = 10240
D = 128
E_EDGES = 320000
NC = 2
NS = 16
NW = NC * NS
EPW = E_EDGES // NW
SB = 2000
NBLK = EPW // SB
BCHUNKS = SB // 16
NBUF = 5
BOUT = BCHUNKS // NBUF
DEG_K = 25
DEG_OUT = BCHUNKS // DEG_K
ROWS_PT = NPAD // NS
ROW_BLK = 1000


def _matmul_body(x_ref, w_ref, o_ref):
    o_ref[...] = jnp.dot(x_ref[...], w_ref[...],
                         preferred_element_type=jnp.float32)


def _xw(x, W):
    return pl.pallas_call(
        _matmul_body,
        grid=(N_NODES // ROW_BLK,),
        in_specs=[
            pl.BlockSpec((ROW_BLK, D), lambda i: (i, 0)),
            pl.BlockSpec((D, D), lambda i: (0, 0)),
        ],
        out_specs=pl.BlockSpec((ROW_BLK, D), lambda i: (i, 0)),
        out_shape=jax.ShapeDtypeStruct((N_NODES, D), jnp.float32),
    )(x, W)


def _epilogue_body(a0_ref, a1_ref, dinv_ref, xw_ref, b_ref, o_ref):
    dinv = dinv_ref[...]
    acc = a0_ref[...] + a1_ref[...]
    out = dinv * acc + (dinv * dinv) * xw_ref[...] + b_ref[...]
    o_ref[...] = jnp.maximum(out, 0.0)


def _epilogue(a0, a1, dinv, xw, b):
    blk = lambda i: (i, 0)
    return pl.pallas_call(
        _epilogue_body,
        grid=(N_NODES // ROW_BLK,),
        in_specs=[
            pl.BlockSpec((ROW_BLK, D), blk),
            pl.BlockSpec((ROW_BLK, D), blk),
            pl.BlockSpec((ROW_BLK, 1), blk),
            pl.BlockSpec((ROW_BLK, D), blk),
            pl.BlockSpec((1, D), lambda i: (0, 0)),
        ],
        out_specs=pl.BlockSpec((ROW_BLK, D), blk),
        out_shape=jax.ShapeDtypeStruct((N_NODES, D), jnp.float32),
    )(a0, a1, dinv.reshape(N_NODES, 1), xw, b.reshape(1, D))


def _deg_body(dst_hbm, w_hbm, z1_hbm,
              degp_out,
              dstb_v, wb_v,
              deg_sp, degsem):
    c = lax.axis_index("c")
    s = lax.axis_index("s")
    wid = s * NC + c
    base = s * ROWS_PT

    pltpu.sync_copy(z1_hbm, deg_sp.at[pl.ds(base, ROWS_PT)])
    plsc.subcore_barrier()

    @pl.loop(0, NBLK)
    def _(blk):
        off = wid * EPW + blk * SB
        pltpu.sync_copy(dst_hbm.at[pl.ds(off, SB)], dstb_v)
        pltpu.sync_copy(w_hbm.at[pl.ds(off, SB)], wb_v)

        @pl.loop(0, DEG_OUT)
        def _(j0):
            for jj in range(DEG_K):
                j = j0 * DEG_K + jj
                dv = dstb_v[pl.ds(j * 16, 16)]
                pltpu.async_copy(wb_v.at[pl.ds(j * 16, 16)],
                                 deg_sp.at[dv], degsem, add=True)
            for jj in range(DEG_K):
                j = j0 * DEG_K + jj
                dv = dstb_v[pl.ds(j * 16, 16)]
                pltpu.make_async_copy(wb_v.at[pl.ds(j * 16, 16)],
                                      deg_sp.at[dv], degsem).wait()

    plsc.subcore_barrier()
    pltpu.sync_copy(deg_sp.at[pl.ds(base, ROWS_PT)],
                    degp_out.at[c, pl.ds(base, ROWS_PT)])


_deg_kernel = functools.partial(
    pl.kernel,
    out_type=jax.ShapeDtypeStruct((NC, NPAD), jnp.float32),
    mesh=plsc.VectorSubcoreMesh(core_axis_name="c", subcore_axis_name="s"),
    compiler_params=pltpu.CompilerParams(needs_layout_passes=False),
    scratch_types=(
        pltpu.VMEM((SB,), jnp.int32),
        pltpu.VMEM((SB,), jnp.float32),
        pltpu.VMEM_SHARED((NPAD,), jnp.float32),
        pltpu.SemaphoreType.DMA,
    ),
)(_deg_body)


def _splat_lane(a, e):
    return lax.gather(
        a, jnp.full((16, 1), e, jnp.int32),
        lax.GatherDimensionNumbers(
            offset_dims=(), collapsed_slice_dims=(0,),
            start_index_map=(0,)),
        (1,), mode=lax.GatherScatterMode.PROMISE_IN_BOUNDS)


def _newton_rsqrt(d):
    i = lax.bitcast_convert_type(d, jnp.int32)
    i = jnp.int32(0x5F3759DF) - lax.shift_right_logical(i, 1)
    y = lax.bitcast_convert_type(i, jnp.float32)
    half_d = 0.5 * d
    for _ in range(3):
        y = y * (1.5 - half_d * y * y)
    return y


def _main_body(xw_hbm, src_hbm, dst_hbm, w_hbm, degp_hbm, z2_hbm,
               acc_out, dinv_out,
               srcb_v, dstb_v, wb_v, dinv_v, degl_v, degl2_v,
               gbuf, sbuf,
               acc_sp, dinv_sp,
               gsem, ssem):
    c = lax.axis_index("c")
    s = lax.axis_index("s")
    wid = s * NC + c
    base = s * ROWS_PT

    pltpu.sync_copy(z2_hbm, acc_sp.at[pl.ds(base, ROWS_PT)])

    pltpu.sync_copy(degp_hbm.at[0, pl.ds(base, ROWS_PT)], degl_v)
    pltpu.sync_copy(degp_hbm.at[1, pl.ds(base, ROWS_PT)], degl2_v)

    @pl.loop(0, ROWS_PT // 16)
    def _(i):
        sl = pl.ds(i * 16, 16)
        d = degl_v[sl] + degl2_v[sl] + 1.0
        degl_v[sl] = _newton_rsqrt(d)

    pltpu.sync_copy(degl_v, dinv_sp.at[pl.ds(base, ROWS_PT)])

    @pl.when(c == 0)
    def _():
        pltpu.sync_copy(degl_v, dinv_out.at[pl.ds(base, ROWS_PT)])
    plsc.subcore_barrier()
    pltpu.sync_copy(dinv_sp, dinv_v)

    def fire_gather(g, b):
        sv = srcb_v[pl.ds(g * 16, 16)]
        pltpu.async_copy(xw_hbm.at[sv], gbuf.at[b], gsem.at[b])

    def wait_gather(g, b):
        sv = srcb_v[pl.ds(g * 16, 16)]
        pltpu.make_async_copy(xw_hbm.at[sv], gbuf.at[b],
                              gsem.at[b]).wait()

    def fire_scatter(g, b):
        dv = dstb_v[pl.ds(g * 16, 16)]
        pltpu.async_copy(sbuf.at[b], acc_sp.at[dv], ssem.at[b], add=True)

    def wait_scatter(g, b):
        dv = dstb_v[pl.ds(g * 16, 16)]
        pltpu.make_async_copy(sbuf.at[b], acc_sp.at[dv],
                              ssem.at[b]).wait()

    @pl.loop(0, NBLK)
    def _(blk):
        off = wid * EPW + blk * SB
        pltpu.sync_copy(src_hbm.at[pl.ds(off, SB)], srcb_v)
        pltpu.sync_copy(dst_hbm.at[pl.ds(off, SB)], dstb_v)
        pltpu.sync_copy(w_hbm.at[pl.ds(off, SB)], wb_v)

        for b in range(NBUF):
            fire_gather(b, b)

        @pl.loop(0, BOUT)
        def _(i):
            for b in range(NBUF):
                g = i * NBUF + b
                wait_gather(g, b)
                sv = srcb_v[pl.ds(g * 16, 16)]
                a = wb_v[pl.ds(g * 16, 16)] * plsc.load_gather(
                    dinv_v, [sv])

                @pl.when(i > 0)
                def _():
                    wait_scatter(g - NBUF, b)

                for e in range(16):
                    ab = _splat_lane(a, e)
                    for cc in range(8):
                        sbuf[b, e, pl.ds(cc * 16, 16)] = (
                            gbuf[b, e, pl.ds(cc * 16, 16)] * ab)
                fire_scatter(g, b)

                @pl.when(i < BOUT - 1)
                def _():
                    fire_gather(g + NBUF, b)

        for b in range(NBUF):
            wait_scatter(BCHUNKS - NBUF + b, b)

    plsc.subcore_barrier()
    pltpu.sync_copy(acc_sp.at[pl.ds(base, ROWS_PT)],
                    acc_out.at[c, pl.ds(base, ROWS_PT)])


_main_kernel = functools.partial(
    pl.kernel,
    out_type=(
        jax.ShapeDtypeStruct((NC, NPAD, D), jnp.float32),
        jax.ShapeDtypeStruct((NPAD,), jnp.float32),
    ),
    mesh=plsc.VectorSubcoreMesh(core_axis_name="c", subcore_axis_name="s"),
    compiler_params=pltpu.CompilerParams(needs_layout_passes=False),
    scratch_types=(
        pltpu.VMEM((SB,), jnp.int32),
        pltpu.VMEM((SB,), jnp.int32),
        pltpu.VMEM((SB,), jnp.float32),
        pltpu.VMEM((NPAD,), jnp.float32),
        pltpu.VMEM((ROWS_PT,), jnp.float32),
        pltpu.VMEM((ROWS_PT,), jnp.float32),
        pltpu.VMEM((NBUF, 16, D), jnp.float32),
        pltpu.VMEM((NBUF, 16, D), jnp.float32),
        pltpu.VMEM_SHARED((NPAD, D), jnp.float32),
        pltpu.VMEM_SHARED((NPAD,), jnp.float32),
        pltpu.SemaphoreType.DMA((NBUF,)),
        pltpu.SemaphoreType.DMA((NBUF,)),
    ),
)(_main_body)


def kernel(x, edge_index, edge_attr, W, b):
    src = edge_index[0]
    dst = edge_index[1]
    xw = _xw(x, W)
    z1 = jnp.zeros((ROWS_PT,), jnp.float32)
    z2 = jnp.zeros((ROWS_PT, D), jnp.float32)
    degp = _deg_kernel(dst, edge_attr, z1)
    acc, dinv = _main_kernel(xw, src, dst, edge_attr, degp, z2)
    return _epilogue(acc[0, :N_NODES], acc[1, :N_NODES],
                     dinv[:N_NODES], xw, b)
